# baseline (speedup 1.0000x reference)
import functools

import jax
import jax.numpy as jnp
from jax import lax
from jax.experimental import pallas as pl
from jax.experimental.pallas import tpu as pltpu
from jax.experimental.pallas import tpu_sc as plsc

_NC = 2
_NS = 16
_NW = _NC * _NS
_CH = 16
_U = 8


def _sc_add(x2, pos2, B, S, D):
    rows_per_w = S // _NW
    chunks = rows_per_w // _CH
    VECS = (_CH * D) // 16
    minor_vecs = D // 16

    mesh = plsc.VectorSubcoreMesh(core_axis_name="c", subcore_axis_name="s")

    @functools.partial(
        pl.kernel,
        mesh=mesh,
        out_type=jax.ShapeDtypeStruct((B * S, D), jnp.float32),
        compiler_params=pltpu.CompilerParams(
            disable_bounds_checks=True,
            disable_semaphore_checks=True,
        ),
        scratch_types=(
            [pltpu.VMEM((2, _CH, D), jnp.float32)]
            + [pltpu.VMEM((_CH, D), jnp.float32) for _ in range(B)]
            + [pltpu.SemaphoreType.DMA((2,))]
            + [pltpu.SemaphoreType.DMA((B,))] * 2
        ),
    )
    def k(x_hbm, pos_hbm, out_hbm, pos_buf, *rest):
        x_bufs = rest[0:B]
        pos_sem, in_sem, out_sem = rest[B:B + 3]

        wid = lax.axis_index("s") * _NC + lax.axis_index("c")
        base_row = wid * rows_per_w

        def pos_slice(c):
            return pos_hbm.at[pl.ds(pl.multiple_of(base_row + c * _CH, _CH), _CH), :]

        def x_slice(hbm, c, b):
            row = pl.multiple_of(b * S + base_row + c * _CH, _CH)
            return hbm.at[pl.ds(row, _CH), :]

        def issue_pos(c, ps):
            pltpu.async_copy(pos_slice(c), pos_buf.at[ps], pos_sem.at[ps])

        def wait_pos(c, ps):
            pltpu.make_async_copy(pos_slice(c), pos_buf.at[ps],
                                  pos_sem.at[ps]).wait()

        def issue_in(c, b, slot):
            pltpu.async_copy(x_slice(x_hbm, c, b), x_bufs[slot],
                             in_sem.at[slot])

        def wait_in(c, b, slot):
            pltpu.make_async_copy(x_slice(x_hbm, c, b), x_bufs[slot],
                                  in_sem.at[slot]).wait()

        def issue_out(c, b, slot):
            pltpu.async_copy(x_bufs[slot], x_slice(out_hbm, c, b),
                             out_sem.at[slot])

        def wait_out(c, b, slot):
            pltpu.make_async_copy(x_bufs[slot], x_slice(out_hbm, c, b),
                                  out_sem.at[slot]).wait()

        issue_pos(0, 0)
        issue_in(0, 0, 0)
        issue_in(0, 1, 1)

        def chunk_body(t, carry):
            ps = lax.rem(t, 2)

            @pl.when(t + 1 < chunks)
            def _pos_prefetch():
                issue_pos(t + 1, lax.rem(t + 1, 2))

            wait_pos(t, ps)

            for b in range(B):
                if b < 2:
                    @pl.when(t >= 1)
                    def _recycle(b=b):
                        wait_out(t - 1, b + 2, b + 2)

                    issue_in(t, b + 2, b + 2)
                else:
                    @pl.when(t + 1 < chunks)
                    def _prefetch(b=b):
                        wait_out(t, b - 2, b - 2)
                        issue_in(t + 1, b - 2, b - 2)

                wait_in(t, b, b)

                @plsc.parallel_loop(0, VECS, step=1, unroll=_U)
                def add_u(i, b=b):
                    r = i // minor_vecs
                    sl = pl.ds((i % minor_vecs) * 16, 16)
                    x_bufs[b][r, sl] = x_bufs[b][r, sl] + pos_buf[ps, r, sl]

                issue_out(t, b, b)
            return carry

        lax.fori_loop(0, chunks, chunk_body, 0)

        for b in range(B):
            wait_out(chunks - 1, b, b)

    return k(x2, pos2)


def kernel(x, pos_table):
    B, S, D = x.shape
    out2 = _sc_add(x.reshape(B * S, D), pos_table[:S], B, S, D)
    return out2.reshape(B, S, D)


# speedup vs baseline: 1.0054x; 1.0054x over previous
import functools

import jax
import jax.numpy as jnp
from jax import lax
from jax.experimental import pallas as pl
from jax.experimental.pallas import tpu as pltpu
from jax.experimental.pallas import tpu_sc as plsc

_NC = 2
_NS = 16
_NW = _NC * _NS
_CH = 16
_U = 8
_K = 5
_P = 2


def _sc_add(x2, pos2, B, S, D):
    rows_per_w = S // _NW
    chunks = rows_per_w // _CH
    NJ = chunks * B
    VECS = (_CH * D) // 16
    minor_vecs = D // 16

    mesh = plsc.VectorSubcoreMesh(core_axis_name="c", subcore_axis_name="s")

    @functools.partial(
        pl.kernel,
        mesh=mesh,
        out_type=jax.ShapeDtypeStruct((B * S, D), jnp.float32),
        compiler_params=pltpu.CompilerParams(
            disable_bounds_checks=True,
            disable_semaphore_checks=True,
        ),
        scratch_types=[
            pltpu.VMEM((2, _CH, D), jnp.float32),
            pltpu.VMEM((_K, _CH, D), jnp.float32),
            pltpu.SemaphoreType.DMA((2,)),
            pltpu.SemaphoreType.DMA((_K,)),
            pltpu.SemaphoreType.DMA((_K,)),
        ],
    )
    def k(x_hbm, pos_hbm, out_hbm, pos_buf, x_buf, pos_sem, in_sem, out_sem):
        wid = lax.axis_index("s") * _NC + lax.axis_index("c")
        base_row = wid * rows_per_w

        def pos_slice(c):
            return pos_hbm.at[pl.ds(pl.multiple_of(base_row + c * _CH, _CH), _CH), :]

        def x_slice(hbm, c, b):
            row = pl.multiple_of(b * S + base_row + c * _CH, _CH)
            return hbm.at[pl.ds(row, _CH), :]

        def issue_pos(c, ps):
            pltpu.async_copy(pos_slice(c), pos_buf.at[ps], pos_sem.at[ps])

        def wait_pos(c, ps):
            pltpu.make_async_copy(pos_slice(c), pos_buf.at[ps],
                                  pos_sem.at[ps]).wait()

        def issue_in(c, b, slot):
            pltpu.async_copy(x_slice(x_hbm, c, b), x_buf.at[slot],
                             in_sem.at[slot])

        def wait_in(c, b, slot):
            pltpu.make_async_copy(x_slice(x_hbm, c, b), x_buf.at[slot],
                                  in_sem.at[slot]).wait()

        def issue_out(c, b, slot):
            pltpu.async_copy(x_buf.at[slot], x_slice(out_hbm, c, b),
                             out_sem.at[slot])

        def wait_out(c, b, slot):
            pltpu.make_async_copy(x_buf.at[slot], x_slice(out_hbm, c, b),
                                  out_sem.at[slot]).wait()

        issue_pos(0, 0)
        for j in range(_P):
            issue_in(j // B, j % B, j % _K)

        def job_body(j, carry):
            c = lax.div(j, B)
            b = lax.rem(j, B)
            slot = lax.rem(j, _K)
            ps = lax.rem(c, 2)

            @pl.when(b == 0)
            def _pos():
                @pl.when(c + 1 < chunks)
                def _():
                    issue_pos(c + 1, lax.rem(c + 1, 2))
                wait_pos(c, ps)

            nj = j + _P

            @pl.when(nj < NJ)
            def _prefetch():
                nslot = lax.rem(nj, _K)
                pj = nj - _K

                @pl.when(pj >= 0)
                def _recycle():
                    wait_out(lax.div(pj, B), lax.rem(pj, B), lax.rem(pj, _K))

                issue_in(lax.div(nj, B), lax.rem(nj, B), nslot)

            wait_in(c, b, slot)

            @plsc.parallel_loop(0, VECS, step=1, unroll=_U)
            def add_u(i):
                r = i // minor_vecs
                sl = pl.ds((i % minor_vecs) * 16, 16)
                x_buf[slot, r, sl] = x_buf[slot, r, sl] + pos_buf[ps, r, sl]

            issue_out(c, b, slot)
            return carry

        lax.fori_loop(0, NJ, job_body, 0)

        for j in range(max(NJ - _K, 0), NJ):
            wait_out(j // B, j % B, j % _K)

    return k(x2, pos2)


def kernel(x, pos_table):
    B, S, D = x.shape
    out2 = _sc_add(x.reshape(B * S, D), pos_table[:S], B, S, D)
    return out2.reshape(B, S, D)


# speedup vs baseline: 1.0056x; 1.0001x over previous
import functools

import jax
import jax.numpy as jnp
from jax import lax
from jax.experimental import pallas as pl
from jax.experimental.pallas import tpu as pltpu
from jax.experimental.pallas import tpu_sc as plsc

_NC = 2
_NS = 16
_NW = _NC * _NS
_CH = 16
_U = 8
_K = 4
_P = 2


def _sc_add(x2, pos2, B, S, D):
    rows_per_w = S // _NW
    chunks = rows_per_w // _CH
    NJ = chunks * B
    VECS = (_CH * D) // 16
    minor_vecs = D // 16

    mesh = plsc.VectorSubcoreMesh(core_axis_name="c", subcore_axis_name="s")

    @functools.partial(
        pl.kernel,
        mesh=mesh,
        out_type=jax.ShapeDtypeStruct((B * S, D), jnp.float32),
        compiler_params=pltpu.CompilerParams(
            disable_bounds_checks=True,
            disable_semaphore_checks=True,
        ),
        scratch_types=[
            pltpu.VMEM((2, _CH, D), jnp.float32),
            pltpu.VMEM((_K, _CH, D), jnp.float32),
            pltpu.SemaphoreType.DMA((2,)),
            pltpu.SemaphoreType.DMA((_K,)),
            pltpu.SemaphoreType.DMA((_K,)),
        ],
    )
    def k(x_hbm, pos_hbm, out_hbm, pos_buf, x_buf, pos_sem, in_sem, out_sem):
        wid = lax.axis_index("s") * _NC + lax.axis_index("c")
        base_row = wid * rows_per_w

        def pos_slice(c):
            return pos_hbm.at[pl.ds(pl.multiple_of(base_row + c * _CH, _CH), _CH), :]

        def x_slice(hbm, c, b):
            row = pl.multiple_of(b * S + base_row + c * _CH, _CH)
            return hbm.at[pl.ds(row, _CH), :]

        def issue_pos(c, ps):
            pltpu.async_copy(pos_slice(c), pos_buf.at[ps], pos_sem.at[ps])

        def wait_pos(c, ps):
            pltpu.make_async_copy(pos_slice(c), pos_buf.at[ps],
                                  pos_sem.at[ps]).wait()

        def issue_in(c, b, slot):
            pltpu.async_copy(x_slice(x_hbm, c, b), x_buf.at[slot],
                             in_sem.at[slot])

        def wait_in(c, b, slot):
            pltpu.make_async_copy(x_slice(x_hbm, c, b), x_buf.at[slot],
                                  in_sem.at[slot]).wait()

        def issue_out(c, b, slot):
            pltpu.async_copy(x_buf.at[slot], x_slice(out_hbm, c, b),
                             out_sem.at[slot])

        def wait_out(c, b, slot):
            pltpu.make_async_copy(x_buf.at[slot], x_slice(out_hbm, c, b),
                                  out_sem.at[slot]).wait()

        issue_pos(0, 0)
        for j in range(_P):
            issue_in(j // B, j % B, j % _K)

        def job_body(j, carry):
            c = lax.div(j, B)
            b = lax.rem(j, B)
            slot = lax.rem(j, _K)
            ps = lax.rem(c, 2)

            @pl.when(b == 0)
            def _pos():
                @pl.when(c + 1 < chunks)
                def _():
                    issue_pos(c + 1, lax.rem(c + 1, 2))
                wait_pos(c, ps)

            nj = j + _P

            @pl.when(nj < NJ)
            def _prefetch():
                nslot = lax.rem(nj, _K)
                pj = nj - _K

                @pl.when(pj >= 0)
                def _recycle():
                    wait_out(lax.div(pj, B), lax.rem(pj, B), lax.rem(pj, _K))

                issue_in(lax.div(nj, B), lax.rem(nj, B), nslot)

            wait_in(c, b, slot)

            @plsc.parallel_loop(0, VECS, step=1, unroll=_U)
            def add_u(i):
                r = i // minor_vecs
                sl = pl.ds((i % minor_vecs) * 16, 16)
                x_buf[slot, r, sl] = x_buf[slot, r, sl] + pos_buf[ps, r, sl]

            issue_out(c, b, slot)
            return carry

        lax.fori_loop(0, NJ, job_body, 0)

        for j in range(max(NJ - _K, 0), NJ):
            wait_out(j // B, j % B, j % _K)

    return k(x2, pos2)


def kernel(x, pos_table):
    B, S, D = x.shape
    out2 = _sc_add(x.reshape(B * S, D), pos_table[:S], B, S, D)
    return out2.reshape(B, S, D)
